# baseline (speedup 1.0000x reference)
import functools

import jax
import jax.numpy as jnp
from jax import lax
from jax.experimental import pallas as pl
from jax.experimental.pallas import tpu as pltpu
from jax.experimental.pallas import tpu_sc as plsc

BATCH = 4096
HIST = 50
EMBED_DIM = 64
VOCAB = 1_000_000
TOTAL = BATCH * HIST


TC_COLS = 2048
HALF = 512_000
TC_STEPS = HALF // TC_COLS
ROWS_VIEW = 2 * HALF


def _transpose_kernel(lo_ref, hi_ref, dst_ref):
    eye = jnp.eye(EMBED_DIM, dtype=jnp.float32)
    dim = (((0,), (0,)), ((), ()))
    dst_ref[:, 0:EMBED_DIM] = lax.dot_general(
        lo_ref[...], eye, dim, precision=lax.Precision.HIGHEST,
        preferred_element_type=jnp.float32,
    )
    dst_ref[:, EMBED_DIM : 2 * EMBED_DIM] = lax.dot_general(
        hi_ref[...], eye, dim, precision=lax.Precision.HIGHEST,
        preferred_element_type=jnp.float32,
    )


@jax.jit
def _transpose_tc(t64):
    return pl.pallas_call(
        _transpose_kernel,
        grid=(TC_STEPS,),
        in_specs=[
            pl.BlockSpec((EMBED_DIM, TC_COLS), lambda i: (0, i)),
            pl.BlockSpec(
                (EMBED_DIM, TC_COLS),
                lambda i: (0, jnp.minimum(i + TC_STEPS, VOCAB // TC_COLS)),
            ),
        ],
        out_specs=pl.BlockSpec((TC_COLS, 2 * EMBED_DIM), lambda i: (i, 0)),
        out_shape=jax.ShapeDtypeStruct((HALF, 2 * EMBED_DIM), jnp.float32),
    )(t64, t64)


_info = plsc.get_sparse_core_info()
NC, NS = _info.num_cores, _info.num_subcores
NW = NC * NS
ROWS_PER_W = TOTAL // NW
CHUNK = 640
CHUNKS_PER_W = ROWS_PER_W // CHUNK
NBUF = 2
LANES = 16


def _embed_kernel(table_hbm, idx_hbm, out_hbm, idx_v, bufs, gsems, wsems):
    wid = lax.axis_index("s") * NC + lax.axis_index("c")
    pltpu.sync_copy(idx_hbm.at[wid], idx_v)
    out_base = wid * ROWS_PER_W

    def remap(row, carry):
        for k in range(CHUNK // LANES):
            v = idx_v[row, k * LANES : (k + 1) * LANES]
            idx_v[row, k * LANES : (k + 1) * LANES] = jnp.where(
                v < HALF, 2 * v, 2 * v - (ROWS_VIEW - 1)
            )
        return carry

    lax.fori_loop(0, CHUNKS_PER_W, remap, 0)

    def gather(j):
        return pltpu.make_async_copy(
            table_hbm.at[idx_v.at[j]], bufs.at[j % NBUF], gsems.at[j % NBUF]
        )

    def write(j):
        return pltpu.make_async_copy(
            bufs.at[j % NBUF],
            out_hbm.at[pl.ds(out_base + j * CHUNK, CHUNK)],
            wsems.at[j % NBUF],
        )

    gather(0).start()
    gather(1).start()
    for j in range(CHUNKS_PER_W):
        gather(j).wait()
        write(j).start()
        write(j).wait()
        if j + NBUF < CHUNKS_PER_W:
            gather(j + NBUF).start()


@jax.jit
def _embed(idx2d, table):
    mesh = plsc.VectorSubcoreMesh(core_axis_name="c", subcore_axis_name="s")
    return pl.kernel(
        _embed_kernel,
        out_type=jax.ShapeDtypeStruct((TOTAL, EMBED_DIM), jnp.float32),
        mesh=mesh,
        scratch_types=[
            pltpu.VMEM((CHUNKS_PER_W, CHUNK), jnp.int32),
            pltpu.VMEM((NBUF, CHUNK, EMBED_DIM), jnp.float32),
            pltpu.SemaphoreType.DMA((NBUF,)),
            pltpu.SemaphoreType.DMA((NBUF,)),
        ],
        compiler_params=pltpu.CompilerParams(use_tc_tiling_on_sc=False),
    )(table, idx2d)


def kernel(inputs, table):
    pair_rows = _transpose_tc(table.T)
    tbl = pair_rows.reshape(ROWS_VIEW, EMBED_DIM)
    idx2d = inputs.T.reshape(NW, CHUNKS_PER_W, CHUNK).astype(jnp.int32)
    out = _embed(idx2d, tbl)
    return out.reshape(HIST, BATCH, EMBED_DIM).transpose(1, 0, 2)


# speedup vs baseline: 1.6197x; 1.6197x over previous
import functools

import jax
import jax.numpy as jnp
from jax import lax
from jax.experimental import pallas as pl
from jax.experimental.pallas import tpu as pltpu
from jax.experimental.pallas import tpu_sc as plsc

BATCH = 4096
HIST = 50
EMBED_DIM = 64
VOCAB = 1_000_000
TOTAL = BATCH * HIST


TC_COLS = 2048
HALF = 512_000
TC_STEPS = HALF // TC_COLS
ROWS_VIEW = 2 * HALF


def _transpose_kernel(lo_ref, hi_ref, dst_ref):
    x = jnp.concatenate([lo_ref[...], hi_ref[...]], axis=0)
    dst_ref[...] = x.T


@jax.jit
def _transpose_tc(t64):
    return pl.pallas_call(
        _transpose_kernel,
        grid=(TC_STEPS,),
        in_specs=[
            pl.BlockSpec((EMBED_DIM, TC_COLS), lambda i: (0, i)),
            pl.BlockSpec(
                (EMBED_DIM, TC_COLS),
                lambda i: (0, jnp.minimum(i + TC_STEPS, VOCAB // TC_COLS)),
            ),
        ],
        out_specs=pl.BlockSpec((TC_COLS, 2 * EMBED_DIM), lambda i: (i, 0)),
        out_shape=jax.ShapeDtypeStruct((HALF, 2 * EMBED_DIM), jnp.float32),
    )(t64, t64)


_info = plsc.get_sparse_core_info()
NC, NS = _info.num_cores, _info.num_subcores
NW = NC * NS
ROWS_PER_W = TOTAL // NW
CHUNK = 640
CHUNKS_PER_W = ROWS_PER_W // CHUNK
NBUF = 2
LANES = 16


def _embed_kernel(table_hbm, idx_hbm, out_hbm, idx_v, bufs, gsems, wsems):
    wid = lax.axis_index("s") * NC + lax.axis_index("c")
    pltpu.sync_copy(idx_hbm.at[wid], idx_v)
    out_base = wid * ROWS_PER_W

    def remap(row, carry):
        for k in range(CHUNK // LANES):
            v = idx_v[row, k * LANES : (k + 1) * LANES]
            idx_v[row, k * LANES : (k + 1) * LANES] = jnp.where(
                v < HALF, 2 * v, 2 * v - (ROWS_VIEW - 1)
            )
        return carry

    lax.fori_loop(0, CHUNKS_PER_W, remap, 0)

    def gather(j):
        return pltpu.make_async_copy(
            table_hbm.at[idx_v.at[j]], bufs.at[j % NBUF], gsems.at[j % NBUF]
        )

    def write(j):
        return pltpu.make_async_copy(
            bufs.at[j % NBUF],
            out_hbm.at[pl.ds(out_base + j * CHUNK, CHUNK)],
            wsems.at[j % NBUF],
        )

    gather(0).start()
    gather(1).start()
    for j in range(CHUNKS_PER_W):
        gather(j).wait()
        write(j).start()
        write(j).wait()
        if j + NBUF < CHUNKS_PER_W:
            gather(j + NBUF).start()


@jax.jit
def _embed(idx2d, table):
    mesh = plsc.VectorSubcoreMesh(core_axis_name="c", subcore_axis_name="s")
    return pl.kernel(
        _embed_kernel,
        out_type=jax.ShapeDtypeStruct((TOTAL, EMBED_DIM), jnp.float32),
        mesh=mesh,
        scratch_types=[
            pltpu.VMEM((CHUNKS_PER_W, CHUNK), jnp.int32),
            pltpu.VMEM((NBUF, CHUNK, EMBED_DIM), jnp.float32),
            pltpu.SemaphoreType.DMA((NBUF,)),
            pltpu.SemaphoreType.DMA((NBUF,)),
        ],
        compiler_params=pltpu.CompilerParams(use_tc_tiling_on_sc=False),
    )(table, idx2d)


def kernel(inputs, table):
    pair_rows = _transpose_tc(table.T)
    tbl = pair_rows.reshape(ROWS_VIEW, EMBED_DIM)
    idx2d = inputs.T.reshape(NW, CHUNKS_PER_W, CHUNK).astype(jnp.int32)
    out = _embed(idx2d, tbl)
    return out.reshape(HIST, BATCH, EMBED_DIM).transpose(1, 0, 2)


# speedup vs baseline: 1.9438x; 1.2001x over previous
import functools

import jax
import jax.numpy as jnp
from jax import lax
from jax.experimental import pallas as pl
from jax.experimental.pallas import tpu as pltpu
from jax.experimental.pallas import tpu_sc as plsc

BATCH = 4096
HIST = 50
EMBED_DIM = 64
VOCAB = 1_000_000
TOTAL = BATCH * HIST


TC_COLS = 4096
HALF = 512_000
TC_STEPS = HALF // TC_COLS
ROWS_VIEW = 2 * HALF


def _transpose_kernel(lo_ref, hi_ref, dst_ref):
    x = jnp.concatenate([lo_ref[...], hi_ref[...]], axis=0)
    dst_ref[...] = x.T


@jax.jit
def _transpose_tc(t64):
    return pl.pallas_call(
        _transpose_kernel,
        grid=(TC_STEPS,),
        in_specs=[
            pl.BlockSpec((EMBED_DIM, TC_COLS), lambda i: (0, i)),
            pl.BlockSpec(
                (EMBED_DIM, TC_COLS),
                lambda i: (0, jnp.minimum(i + TC_STEPS, VOCAB // TC_COLS)),
            ),
        ],
        out_specs=pl.BlockSpec((TC_COLS, 2 * EMBED_DIM), lambda i: (i, 0)),
        out_shape=jax.ShapeDtypeStruct((HALF, 2 * EMBED_DIM), jnp.float32),
    )(t64, t64)


_info = plsc.get_sparse_core_info()
NC, NS = _info.num_cores, _info.num_subcores
NW = NC * NS
ROWS_PER_W = TOTAL // NW
CHUNK = 640
CHUNKS_PER_W = ROWS_PER_W // CHUNK
NBUF = 2
LANES = 16


def _embed_kernel(table_hbm, idx_hbm, out_hbm, idx_v, bufs, gsems, wsems):
    wid = lax.axis_index("s") * NC + lax.axis_index("c")
    pltpu.sync_copy(idx_hbm.at[wid], idx_v)
    out_base = wid * ROWS_PER_W

    def remap(row, carry):
        for k in range(CHUNK // LANES):
            v = idx_v[row, k * LANES : (k + 1) * LANES]
            idx_v[row, k * LANES : (k + 1) * LANES] = jnp.where(
                v < HALF, 2 * v, 2 * v - (ROWS_VIEW - 1)
            )
        return carry

    lax.fori_loop(0, CHUNKS_PER_W, remap, 0)

    def gather(j):
        return pltpu.make_async_copy(
            table_hbm.at[idx_v.at[j]], bufs.at[j % NBUF], gsems.at[j % NBUF]
        )

    def write(j):
        return pltpu.make_async_copy(
            bufs.at[j % NBUF],
            out_hbm.at[pl.ds(out_base + j * CHUNK, CHUNK)],
            wsems.at[j % NBUF],
        )

    gather(0).start()
    gather(1).start()
    for j in range(CHUNKS_PER_W):
        gather(j).wait()
        write(j).start()
        write(j).wait()
        if j + NBUF < CHUNKS_PER_W:
            gather(j + NBUF).start()


@jax.jit
def _embed(idx2d, table):
    mesh = plsc.VectorSubcoreMesh(core_axis_name="c", subcore_axis_name="s")
    return pl.kernel(
        _embed_kernel,
        out_type=jax.ShapeDtypeStruct((TOTAL, EMBED_DIM), jnp.float32),
        mesh=mesh,
        scratch_types=[
            pltpu.VMEM((CHUNKS_PER_W, CHUNK), jnp.int32),
            pltpu.VMEM((NBUF, CHUNK, EMBED_DIM), jnp.float32),
            pltpu.SemaphoreType.DMA((NBUF,)),
            pltpu.SemaphoreType.DMA((NBUF,)),
        ],
        compiler_params=pltpu.CompilerParams(use_tc_tiling_on_sc=False),
    )(table, idx2d)


def kernel(inputs, table):
    pair_rows = _transpose_tc(table.T)
    tbl = pair_rows.reshape(ROWS_VIEW, EMBED_DIM)
    idx2d = inputs.T.reshape(NW, CHUNKS_PER_W, CHUNK).astype(jnp.int32)
    out = _embed(idx2d, tbl)
    return out.reshape(HIST, BATCH, EMBED_DIM).transpose(1, 0, 2)


# speedup vs baseline: 2.1070x; 1.0840x over previous
import functools

import jax
import jax.numpy as jnp
from jax import lax
from jax.experimental import pallas as pl
from jax.experimental.pallas import tpu as pltpu
from jax.experimental.pallas import tpu_sc as plsc

BATCH = 4096
HIST = 50
EMBED_DIM = 64
VOCAB = 1_000_000
TOTAL = BATCH * HIST


TC_COLS = 8192
HALF = 63 * TC_COLS
TC_STEPS = HALF // TC_COLS
ROWS_VIEW = 2 * HALF


def _transpose_kernel(lo_ref, hi_ref, dst_ref):
    x = jnp.concatenate([lo_ref[...], hi_ref[...]], axis=0)
    dst_ref[...] = x.T


@jax.jit
def _transpose_tc(t64):
    return pl.pallas_call(
        _transpose_kernel,
        grid=(TC_STEPS,),
        in_specs=[
            pl.BlockSpec((EMBED_DIM, TC_COLS), lambda i: (0, i)),
            pl.BlockSpec(
                (EMBED_DIM, TC_COLS),
                lambda i: (0, jnp.minimum(i + TC_STEPS, VOCAB // TC_COLS)),
            ),
        ],
        out_specs=pl.BlockSpec((TC_COLS, 2 * EMBED_DIM), lambda i: (i, 0)),
        out_shape=jax.ShapeDtypeStruct((HALF, 2 * EMBED_DIM), jnp.float32),
    )(t64, t64)


_info = plsc.get_sparse_core_info()
NC, NS = _info.num_cores, _info.num_subcores
NW = NC * NS
ROWS_PER_W = TOTAL // NW
CHUNK = 640
CHUNKS_PER_W = ROWS_PER_W // CHUNK
NBUF = 2
LANES = 16


def _embed_kernel(table_hbm, idx_hbm, out_hbm, idx_v, bufs, gsems, wsems):
    wid = lax.axis_index("s") * NC + lax.axis_index("c")
    pltpu.sync_copy(idx_hbm.at[wid], idx_v)
    out_base = wid * ROWS_PER_W

    def remap(row, carry):
        for k in range(CHUNK // LANES):
            v = idx_v[row, k * LANES : (k + 1) * LANES]
            idx_v[row, k * LANES : (k + 1) * LANES] = jnp.where(
                v < HALF, 2 * v, 2 * v - (ROWS_VIEW - 1)
            )
        return carry

    lax.fori_loop(0, CHUNKS_PER_W, remap, 0)

    def gather(j):
        return pltpu.make_async_copy(
            table_hbm.at[idx_v.at[j]], bufs.at[j % NBUF], gsems.at[j % NBUF]
        )

    def write(j):
        return pltpu.make_async_copy(
            bufs.at[j % NBUF],
            out_hbm.at[pl.ds(out_base + j * CHUNK, CHUNK)],
            wsems.at[j % NBUF],
        )

    gather(0).start()
    gather(1).start()
    for j in range(CHUNKS_PER_W):
        gather(j).wait()
        write(j).start()
        write(j).wait()
        if j + NBUF < CHUNKS_PER_W:
            gather(j + NBUF).start()


@jax.jit
def _embed(idx2d, table):
    mesh = plsc.VectorSubcoreMesh(core_axis_name="c", subcore_axis_name="s")
    return pl.kernel(
        _embed_kernel,
        out_type=jax.ShapeDtypeStruct((TOTAL, EMBED_DIM), jnp.float32),
        mesh=mesh,
        scratch_types=[
            pltpu.VMEM((CHUNKS_PER_W, CHUNK), jnp.int32),
            pltpu.VMEM((NBUF, CHUNK, EMBED_DIM), jnp.float32),
            pltpu.SemaphoreType.DMA((NBUF,)),
            pltpu.SemaphoreType.DMA((NBUF,)),
        ],
        compiler_params=pltpu.CompilerParams(use_tc_tiling_on_sc=False),
    )(table, idx2d)


def kernel(inputs, table):
    pair_rows = _transpose_tc(table.T)
    tbl = pair_rows.reshape(ROWS_VIEW, EMBED_DIM)
    idx2d = inputs.T.reshape(NW, CHUNKS_PER_W, CHUNK).astype(jnp.int32)
    out = _embed(idx2d, tbl)
    return out.reshape(HIST, BATCH, EMBED_DIM).transpose(1, 0, 2)


# speedup vs baseline: 2.1343x; 1.0129x over previous
import functools

import jax
import jax.numpy as jnp
from jax import lax
from jax.experimental import pallas as pl
from jax.experimental.pallas import tpu as pltpu
from jax.experimental.pallas import tpu_sc as plsc

BATCH = 4096
HIST = 50
EMBED_DIM = 64
VOCAB = 1_000_000
TOTAL = BATCH * HIST


TC_COLS = 16384
HALF = 32 * TC_COLS
TC_STEPS = HALF // TC_COLS
ROWS_VIEW = 2 * HALF


def _transpose_kernel(lo_ref, hi_ref, dst_ref):
    x = jnp.concatenate([lo_ref[...], hi_ref[...]], axis=0)
    dst_ref[...] = x.T


@jax.jit
def _transpose_tc(t64):
    return pl.pallas_call(
        _transpose_kernel,
        grid=(TC_STEPS,),
        in_specs=[
            pl.BlockSpec((EMBED_DIM, TC_COLS), lambda i: (0, i)),
            pl.BlockSpec(
                (EMBED_DIM, TC_COLS),
                lambda i: (0, jnp.minimum(i + TC_STEPS, VOCAB // TC_COLS)),
            ),
        ],
        out_specs=pl.BlockSpec((TC_COLS, 2 * EMBED_DIM), lambda i: (i, 0)),
        out_shape=jax.ShapeDtypeStruct((HALF, 2 * EMBED_DIM), jnp.float32),
    )(t64, t64)


_info = plsc.get_sparse_core_info()
NC, NS = _info.num_cores, _info.num_subcores
NW = NC * NS
ROWS_PER_W = TOTAL // NW
CHUNK = 640
CHUNKS_PER_W = ROWS_PER_W // CHUNK
NBUF = 2
LANES = 16


def _embed_kernel(table_hbm, idx_hbm, out_hbm, idx_v, bufs, gsems, wsems):
    wid = lax.axis_index("s") * NC + lax.axis_index("c")
    pltpu.sync_copy(idx_hbm.at[wid], idx_v)
    out_base = wid * ROWS_PER_W

    def remap(row, carry):
        for k in range(CHUNK // LANES):
            v = idx_v[row, k * LANES : (k + 1) * LANES]
            idx_v[row, k * LANES : (k + 1) * LANES] = jnp.where(
                v < HALF, 2 * v, 2 * v - (ROWS_VIEW - 1)
            )
        return carry

    lax.fori_loop(0, CHUNKS_PER_W, remap, 0)

    def gather(j):
        return pltpu.make_async_copy(
            table_hbm.at[idx_v.at[j]], bufs.at[j % NBUF], gsems.at[j % NBUF]
        )

    def write(j):
        return pltpu.make_async_copy(
            bufs.at[j % NBUF],
            out_hbm.at[pl.ds(out_base + j * CHUNK, CHUNK)],
            wsems.at[j % NBUF],
        )

    gather(0).start()
    gather(1).start()
    for j in range(CHUNKS_PER_W):
        gather(j).wait()
        write(j).start()
        write(j).wait()
        if j + NBUF < CHUNKS_PER_W:
            gather(j + NBUF).start()


@jax.jit
def _embed(idx2d, table):
    mesh = plsc.VectorSubcoreMesh(core_axis_name="c", subcore_axis_name="s")
    return pl.kernel(
        _embed_kernel,
        out_type=jax.ShapeDtypeStruct((TOTAL, EMBED_DIM), jnp.float32),
        mesh=mesh,
        scratch_types=[
            pltpu.VMEM((CHUNKS_PER_W, CHUNK), jnp.int32),
            pltpu.VMEM((NBUF, CHUNK, EMBED_DIM), jnp.float32),
            pltpu.SemaphoreType.DMA((NBUF,)),
            pltpu.SemaphoreType.DMA((NBUF,)),
        ],
        compiler_params=pltpu.CompilerParams(use_tc_tiling_on_sc=False),
    )(table, idx2d)


def kernel(inputs, table):
    pair_rows = _transpose_tc(table.T)
    tbl = pair_rows.reshape(ROWS_VIEW, EMBED_DIM)
    idx2d = inputs.T.reshape(NW, CHUNKS_PER_W, CHUNK).astype(jnp.int32)
    out = _embed(idx2d, tbl)
    return out.reshape(HIST, BATCH, EMBED_DIM).transpose(1, 0, 2)
